# Initial kernel scaffold; baseline (speedup 1.0000x reference)
import jax
import jax.numpy as jnp
from jax.experimental import pallas as pl


def kernel(x, edge_index, W_self, W_neigh, b):
    raise NotImplementedError("write your pallas kernel here")


# speedup vs baseline: 1.9620x; 1.9620x over previous
import functools

import jax
import jax.numpy as jnp
from jax import lax
from jax.experimental import pallas as pl
from jax.experimental.pallas import tpu as pltpu
from jax.experimental.pallas import tpu_sc as plsc

N_NODES = 10000
N_EDGES = 160000
D = 256
NQ = 4
FW = D // NQ

NPAD = 10240
ROWS_PER_SUB = NPAD // 16
EPAD = 163840
CHUNKS = 40
CW = 128

_mesh = plsc.VectorSubcoreMesh(core_axis_name="c", subcore_axis_name="s")


@functools.partial(
    pl.kernel,
    mesh=_mesh,
    compiler_params=pltpu.CompilerParams(use_tc_tiling_on_sc=False),
    out_type=[
        jax.ShapeDtypeStruct((2, NQ, NPAD, FW), jnp.float32),
        jax.ShapeDtypeStruct((2, NPAD, 16), jnp.float32),
    ],
    scratch_types=[
        pltpu.VMEM((CHUNKS, CW), jnp.int32),
        pltpu.VMEM((CHUNKS, CW), jnp.int32),
        pltpu.VMEM((CW, FW), jnp.float32),
        pltpu.VMEM((CW, 16), jnp.float32),
        pltpu.VMEM((ROWS_PER_SUB, 16), jnp.float32),
        pltpu.VMEM_SHARED((NPAD, FW), jnp.float32),
        pltpu.VMEM_SHARED((NPAD, 16), jnp.float32),
        pltpu.SemaphoreType.DMA,
    ],
)
def _sc_aggregate(xq, src_r, dst_r, psum, pdeg,
                  idx_s, idx_d, rows, ones_v, dbuf, acc_sh, deg_sh, sem):
    c = lax.axis_index("c")
    s = lax.axis_index("s")
    base = s * ROWS_PER_SUB

    def _zero_rows(i, _):
        for l in range(FW // 16):
            rows[i, pl.ds(l * 16, 16)] = jnp.zeros((16,), jnp.float32)
        return 0

    def _fill_ones(i, _):
        ones_v[i, :] = jnp.ones((16,), jnp.float32)
        return 0

    lax.fori_loop(0, CW, _zero_rows, 0, unroll=False)
    lax.fori_loop(0, CW, _fill_ones, 0, unroll=False)

    def _fill_dbuf(i, _):
        dbuf[i, :] = jnp.zeros((16,), jnp.float32)
        return 0

    lax.fori_loop(0, ROWS_PER_SUB, _fill_dbuf, 0, unroll=False)

    pltpu.sync_copy(src_r.at[c, s], idx_s)
    pltpu.sync_copy(dst_r.at[c, s], idx_d)

    def _zero_acc(t, _):
        pltpu.sync_copy(rows, acc_sh.at[pl.ds(base + t * CW, CW)])
        return 0

    lax.fori_loop(0, ROWS_PER_SUB // CW, _zero_acc, 0, unroll=False)
    pltpu.sync_copy(dbuf, deg_sh.at[pl.ds(base, ROWS_PER_SUB)])
    plsc.subcore_barrier()

    for q in range(NQ):
        def _edge_chunk(j, _):
            pltpu.async_copy(xq.at[q].at[idx_s.at[j]], rows, sem).wait()
            pltpu.sync_copy(rows, acc_sh.at[idx_d.at[j]], add=True)
            if q == 0:
                pltpu.sync_copy(ones_v, deg_sh.at[idx_d.at[j]], add=True)
            return 0

        lax.fori_loop(0, CHUNKS, _edge_chunk, 0, unroll=False)
        plsc.subcore_barrier()

        def _wb(t, _):
            pltpu.sync_copy(acc_sh.at[pl.ds(base + t * CW, CW)], rows)
            pltpu.sync_copy(rows, psum.at[c, q, pl.ds(base + t * CW, CW)])
            return 0

        lax.fori_loop(0, ROWS_PER_SUB // CW, _wb, 0, unroll=False)

        if q == 0:
            pltpu.sync_copy(deg_sh.at[pl.ds(base, ROWS_PER_SUB)], dbuf)
            pltpu.sync_copy(dbuf, pdeg.at[c, pl.ds(base, ROWS_PER_SUB)])

        if q < NQ - 1:
            def _rezero(t, _):
                for l in range(FW // 16):
                    rows[t, pl.ds(l * 16, 16)] = jnp.zeros((16,), jnp.float32)
                return 0

            lax.fori_loop(0, CW, _rezero, 0, unroll=False)

            def _zero2(t, _):
                pltpu.sync_copy(rows, acc_sh.at[pl.ds(base + t * CW, CW)])
                return 0

            lax.fori_loop(0, ROWS_PER_SUB // CW, _zero2, 0, unroll=False)
            plsc.subcore_barrier()


BLK = 512


def _tc_body(x_ref, p0, p1, d0, d1, wst, wnt, b_ref, o_ref):
    deg = jnp.maximum(d0[:, 0:1] + d1[:, 0:1], 1.0)
    hn = (p0[...] + p1[...]) / deg
    o_ref[...] = (
        jnp.dot(x_ref[...], wst[...], preferred_element_type=jnp.float32)
        + jnp.dot(hn, wnt[...], preferred_element_type=jnp.float32)
        + b_ref[...]
    )


def _tc_combine(xp, p0, p1, d0, d1, wst, wnt, b2d):
    grid = (NPAD // BLK,)
    return pl.pallas_call(
        _tc_body,
        grid=grid,
        in_specs=[
            pl.BlockSpec((BLK, D), lambda i: (i, 0)),
            pl.BlockSpec((BLK, D), lambda i: (i, 0)),
            pl.BlockSpec((BLK, D), lambda i: (i, 0)),
            pl.BlockSpec((BLK, 16), lambda i: (i, 0)),
            pl.BlockSpec((BLK, 16), lambda i: (i, 0)),
            pl.BlockSpec((D, D), lambda i: (0, 0)),
            pl.BlockSpec((D, D), lambda i: (0, 0)),
            pl.BlockSpec((1, D), lambda i: (0, 0)),
        ],
        out_specs=pl.BlockSpec((BLK, D), lambda i: (i, 0)),
        out_shape=jax.ShapeDtypeStruct((NPAD, D), jnp.float32),
    )(xp, p0, p1, d0, d1, wst, wnt, b2d)


def kernel(x, edge_index, W_self, W_neigh, b):
    x = x.astype(jnp.float32)
    src = edge_index[0].astype(jnp.int32)
    dst = edge_index[1].astype(jnp.int32)

    xp = jnp.pad(x, ((0, NPAD - N_NODES), (0, 0)))
    xq = jnp.transpose(xp.reshape(NPAD, NQ, FW), (1, 0, 2))

    pad = jnp.full((EPAD - N_EDGES,), NPAD - 1, jnp.int32)
    src_r = jnp.concatenate([src, pad]).reshape(2, 16, CHUNKS, CW)
    dst_r = jnp.concatenate([dst, pad]).reshape(2, 16, CHUNKS, CW)

    psum, pdeg = _sc_aggregate(xq, src_r, dst_r)

    p0 = jnp.transpose(psum[0], (1, 0, 2)).reshape(NPAD, D)
    p1 = jnp.transpose(psum[1], (1, 0, 2)).reshape(NPAD, D)

    out = _tc_combine(
        xp, p0, p1, pdeg[0], pdeg[1],
        W_self.T, W_neigh.T, b.reshape(1, D),
    )
    return out[:N_NODES]


# speedup vs baseline: 2.6805x; 1.3662x over previous
import functools

import jax
import jax.numpy as jnp
from jax import lax
from jax.experimental import pallas as pl
from jax.experimental.pallas import tpu as pltpu
from jax.experimental.pallas import tpu_sc as plsc

N_NODES = 10000
N_EDGES = 160000
D = 256
NQ = 4
FW = D // NQ

NPAD = 10240
ROWS_PER_SUB = NPAD // 16
EPAD = 163840
CHUNKS = 40
CW = 128
NBUF = 4

_mesh = plsc.VectorSubcoreMesh(core_axis_name="c", subcore_axis_name="s")


@functools.partial(
    pl.kernel,
    mesh=_mesh,
    compiler_params=pltpu.CompilerParams(use_tc_tiling_on_sc=False),
    out_type=[
        jax.ShapeDtypeStruct((2, NQ, NPAD, FW), jnp.float32),
        jax.ShapeDtypeStruct((2, NPAD, 16), jnp.float32),
    ],
    scratch_types=[
        pltpu.VMEM((NQ * CHUNKS, CW), jnp.int32),
        pltpu.VMEM((CHUNKS, CW), jnp.int32),
        [pltpu.VMEM((CW, FW), jnp.float32) for _ in range(NBUF)],
        pltpu.VMEM((CW, FW), jnp.float32),
        pltpu.VMEM((CW, 16), jnp.float32),
        pltpu.VMEM((ROWS_PER_SUB, 16), jnp.float32),
        pltpu.VMEM_SHARED((NPAD, FW), jnp.float32),
        pltpu.VMEM_SHARED((NPAD, 16), jnp.float32),
        [pltpu.SemaphoreType.DMA for _ in range(NBUF)],
        [pltpu.SemaphoreType.DMA for _ in range(NBUF)],
        [pltpu.SemaphoreType.DMA for _ in range(NBUF)],
        pltpu.SemaphoreType.DMA,
    ],
)
def _sc_aggregate(tbl, srcq_r, dst_r, psum, pdeg,
                  idx_s, idx_d, rows, zrows, ones_v, dbuf, acc_sh, deg_sh,
                  sg, ss, sd, swb):
    c = lax.axis_index("c")
    s = lax.axis_index("s")
    base = s * ROWS_PER_SUB

    def _zero_zrows(i, _):
        for l in range(FW // 16):
            zrows[i, pl.ds(l * 16, 16)] = jnp.zeros((16,), jnp.float32)
        return 0

    def _fill_ones(i, _):
        ones_v[i, :] = jnp.ones((16,), jnp.float32)
        return 0

    lax.fori_loop(0, CW, _zero_zrows, 0, unroll=False)
    lax.fori_loop(0, CW, _fill_ones, 0, unroll=False)

    def _fill_dbuf(i, _):
        dbuf[i, :] = jnp.zeros((16,), jnp.float32)
        return 0

    lax.fori_loop(0, ROWS_PER_SUB, _fill_dbuf, 0, unroll=False)

    pltpu.sync_copy(srcq_r.at[c, s], idx_s)
    pltpu.sync_copy(dst_r.at[c, s], idx_d)

    for t in range(ROWS_PER_SUB // CW):
        pltpu.sync_copy(zrows, acc_sh.at[pl.ds(base + t * CW, CW)])
    pltpu.sync_copy(dbuf, deg_sh.at[pl.ds(base, ROWS_PER_SUB)])
    plsc.subcore_barrier()

    for q in range(NQ):
        qbase = q * CHUNKS

        for b in range(NBUF):
            pltpu.async_copy(tbl.at[idx_s.at[qbase + b]], rows[b], sg[b])

        def _ring_block(t, _):
            for b in range(NBUF):
                j = t * NBUF + b
                pltpu.make_async_copy(
                    tbl.at[idx_s.at[qbase]], rows[b], sg[b]).wait()
                pltpu.async_copy(
                    rows[b], acc_sh.at[idx_d.at[j]], ss[b], add=True)
                if q == 0:
                    pltpu.async_copy(
                        ones_v, deg_sh.at[idx_d.at[j]], sd[b], add=True)
                pltpu.make_async_copy(
                    rows[b], acc_sh.at[idx_d.at[0]], ss[b]).wait()
                if q == 0:
                    pltpu.make_async_copy(
                        ones_v, deg_sh.at[idx_d.at[0]], sd[b]).wait()

                @pl.when(t < CHUNKS // NBUF - 1)
                def _():
                    pltpu.async_copy(
                        tbl.at[idx_s.at[qbase + j + NBUF]], rows[b], sg[b])
            return 0

        lax.fori_loop(0, CHUNKS // NBUF, _ring_block, 0, unroll=False)
        plsc.subcore_barrier()

        for t in range(ROWS_PER_SUB // CW):
            b = t % 2
            if t >= 2:
                pltpu.make_async_copy(
                    rows[b], psum.at[c, q, pl.ds(base, CW)], swb).wait()
            pltpu.sync_copy(acc_sh.at[pl.ds(base + t * CW, CW)], rows[b])
            pltpu.async_copy(
                rows[b], psum.at[c, q, pl.ds(base + t * CW, CW)], swb)
        for t in range(2):
            pltpu.make_async_copy(
                rows[t], psum.at[c, q, pl.ds(base, CW)], swb).wait()

        if q == 0:
            pltpu.sync_copy(deg_sh.at[pl.ds(base, ROWS_PER_SUB)], dbuf)
            pltpu.sync_copy(dbuf, pdeg.at[c, pl.ds(base, ROWS_PER_SUB)])

        if q < NQ - 1:
            for t in range(ROWS_PER_SUB // CW):
                pltpu.sync_copy(zrows, acc_sh.at[pl.ds(base + t * CW, CW)])
            plsc.subcore_barrier()


BLK = 2000


def _tc_body(x_ref, ps, dg, wst, wnt, b_ref, o_ref):
    deg = jnp.maximum(dg[0, :, 0:1] + dg[1, :, 0:1], 1.0)
    hn = jnp.concatenate(
        [ps[0, q] + ps[1, q] for q in range(NQ)], axis=1) / deg
    o_ref[...] = (
        jnp.dot(x_ref[...], wst[...], preferred_element_type=jnp.float32)
        + jnp.dot(hn, wnt[...], preferred_element_type=jnp.float32)
        + b_ref[...]
    )


def _tc_combine(x, psum, pdeg, wst, wnt, b2d):
    return pl.pallas_call(
        _tc_body,
        grid=(N_NODES // BLK,),
        in_specs=[
            pl.BlockSpec((BLK, D), lambda i: (i, 0)),
            pl.BlockSpec((2, NQ, BLK, FW), lambda i: (0, 0, i, 0)),
            pl.BlockSpec((2, BLK, 16), lambda i: (0, i, 0)),
            pl.BlockSpec((D, D), lambda i: (0, 0)),
            pl.BlockSpec((D, D), lambda i: (0, 0)),
            pl.BlockSpec((1, D), lambda i: (0, 0)),
        ],
        out_specs=pl.BlockSpec((BLK, D), lambda i: (i, 0)),
        out_shape=jax.ShapeDtypeStruct((N_NODES, D), jnp.float32),
    )(x, psum, pdeg, wst, wnt, b2d)


def kernel(x, edge_index, W_self, W_neigh, b):
    x = x.astype(jnp.float32)
    src = edge_index[0].astype(jnp.int32)
    dst = edge_index[1].astype(jnp.int32)

    tbl = x.reshape(N_NODES * NQ, FW)

    npad_e = EPAD - N_EDGES
    src_p = jnp.concatenate(
        [src, jnp.zeros((npad_e,), jnp.int32)]).reshape(2, 16, CHUNKS, CW)
    dst_p = jnp.concatenate(
        [dst, jnp.full((npad_e,), NPAD - 1, jnp.int32)]
    ).reshape(2, 16, CHUNKS, CW)
    srcq_r = (
        src_p[:, :, None, :, :] * NQ
        + jnp.arange(NQ, dtype=jnp.int32)[None, None, :, None, None]
    ).reshape(2, 16, NQ * CHUNKS, CW)

    psum, pdeg = _sc_aggregate(tbl, srcq_r, dst_p)

    return _tc_combine(
        x, psum, pdeg, W_self.T, W_neigh.T, b.reshape(1, D),
    )


# speedup vs baseline: 2.7460x; 1.0244x over previous
import functools

import jax
import jax.numpy as jnp
from jax import lax
from jax.experimental import pallas as pl
from jax.experimental.pallas import tpu as pltpu
from jax.experimental.pallas import tpu_sc as plsc

N_NODES = 10000
N_EDGES = 160000
D = 256
NQ = 4
FW = D // NQ

NPAD = 10240
ROWS_PER_SUB = NPAD // 16
EPAD = 163840
CHUNKS = 40
CW = 128
NBUF = 4

_mesh = plsc.VectorSubcoreMesh(core_axis_name="c", subcore_axis_name="s")


@functools.partial(
    pl.kernel,
    mesh=_mesh,
    compiler_params=pltpu.CompilerParams(use_tc_tiling_on_sc=False),
    out_type=[
        jax.ShapeDtypeStruct((2, NQ, NPAD, FW), jnp.float32),
        jax.ShapeDtypeStruct((2, NPAD, 16), jnp.float32),
    ],
    scratch_types=[
        pltpu.VMEM((NQ * CHUNKS, CW), jnp.int32),
        pltpu.VMEM((CHUNKS, CW), jnp.int32),
        [pltpu.VMEM((CW, FW), jnp.float32) for _ in range(NBUF)],
        pltpu.VMEM((CW, FW), jnp.float32),
        pltpu.VMEM((CW, 16), jnp.float32),
        pltpu.VMEM((ROWS_PER_SUB, 16), jnp.float32),
        pltpu.VMEM_SHARED((NPAD, FW), jnp.float32),
        pltpu.VMEM_SHARED((NPAD, 16), jnp.float32),
        [pltpu.SemaphoreType.DMA for _ in range(NBUF)],
        [pltpu.SemaphoreType.DMA for _ in range(NBUF)],
        [pltpu.SemaphoreType.DMA for _ in range(NBUF)],
        pltpu.SemaphoreType.DMA,
    ],
)
def _sc_aggregate(tbl, srcq_r, dst_r, psum, pdeg,
                  idx_s, idx_d, rows, zrows, ones_v, dbuf, acc_sh, deg_sh,
                  sg, ss, sd, swb):
    c = lax.axis_index("c")
    s = lax.axis_index("s")
    base = s * ROWS_PER_SUB

    def _zero_zrows(i, _):
        for l in range(FW // 16):
            zrows[i, pl.ds(l * 16, 16)] = jnp.zeros((16,), jnp.float32)
        return 0

    def _fill_ones(i, _):
        ones_v[i, :] = jnp.ones((16,), jnp.float32)
        return 0

    lax.fori_loop(0, CW, _zero_zrows, 0, unroll=False)
    lax.fori_loop(0, CW, _fill_ones, 0, unroll=False)

    def _fill_dbuf(i, _):
        dbuf[i, :] = jnp.zeros((16,), jnp.float32)
        return 0

    lax.fori_loop(0, ROWS_PER_SUB, _fill_dbuf, 0, unroll=False)

    pltpu.sync_copy(srcq_r.at[c, s], idx_s)
    pltpu.sync_copy(dst_r.at[c, s], idx_d)

    for t in range(ROWS_PER_SUB // CW):
        pltpu.sync_copy(zrows, acc_sh.at[pl.ds(base + t * CW, CW)])
    pltpu.sync_copy(dbuf, deg_sh.at[pl.ds(base, ROWS_PER_SUB)])
    plsc.subcore_barrier()

    for q in range(NQ):
        qbase = q * CHUNKS

        for b in range(NBUF):
            pltpu.async_copy(tbl.at[idx_s.at[qbase + b]], rows[b], sg[b])

        def _ring_block(t, _):
            for b in range(NBUF):
                j = t * NBUF + b
                pltpu.make_async_copy(
                    tbl.at[idx_s.at[qbase]], rows[b], sg[b]).wait()
                pltpu.async_copy(
                    rows[b], acc_sh.at[idx_d.at[j]], ss[b], add=True)
                if q == 0:
                    pltpu.async_copy(
                        ones_v, deg_sh.at[idx_d.at[j]], sd[b], add=True)
                pltpu.make_async_copy(
                    rows[b], acc_sh.at[idx_d.at[0]], ss[b]).wait()
                if q == 0:
                    pltpu.make_async_copy(
                        ones_v, deg_sh.at[idx_d.at[0]], sd[b]).wait()

                @pl.when(t < CHUNKS // NBUF - 1)
                def _():
                    pltpu.async_copy(
                        tbl.at[idx_s.at[qbase + j + NBUF]], rows[b], sg[b])
            return 0

        lax.fori_loop(0, CHUNKS // NBUF, _ring_block, 0, unroll=False)
        plsc.subcore_barrier()

        for t in range(ROWS_PER_SUB // CW):
            b = t % 2
            if t >= 2:
                pltpu.make_async_copy(
                    rows[b], psum.at[c, q, pl.ds(base, CW)], swb).wait()
            pltpu.sync_copy(acc_sh.at[pl.ds(base + t * CW, CW)], rows[b])
            pltpu.async_copy(
                rows[b], psum.at[c, q, pl.ds(base + t * CW, CW)], swb)
        for t in range(2):
            pltpu.make_async_copy(
                rows[t], psum.at[c, q, pl.ds(base, CW)], swb).wait()

        if q == 0:
            pltpu.sync_copy(deg_sh.at[pl.ds(base, ROWS_PER_SUB)], dbuf)
            pltpu.sync_copy(dbuf, pdeg.at[c, pl.ds(base, ROWS_PER_SUB)])

        if q < NQ - 1:
            for t in range(ROWS_PER_SUB // CW):
                pltpu.sync_copy(zrows, acc_sh.at[pl.ds(base + t * CW, CW)])
            plsc.subcore_barrier()


BLK = 2000


def _tc_body(x_ref, ps, dg, wst, wnt, b_ref, o_ref):
    deg = jnp.maximum(dg[0, :, 0:1] + dg[1, :, 0:1], 1.0)
    hn = jnp.concatenate(
        [ps[0, q] + ps[1, q] for q in range(NQ)], axis=1) / deg
    o_ref[...] = (
        jnp.dot(x_ref[...], wst[...], preferred_element_type=jnp.float32)
        + jnp.dot(hn, wnt[...], preferred_element_type=jnp.float32)
        + b_ref[...]
    )


def _tc_combine(x, psum, pdeg, wst, wnt, b2d):
    return pl.pallas_call(
        _tc_body,
        grid=(N_NODES // BLK,),
        in_specs=[
            pl.BlockSpec((BLK, D), lambda i: (i, 0)),
            pl.BlockSpec((2, NQ, BLK, FW), lambda i: (0, 0, i, 0)),
            pl.BlockSpec((2, BLK, 16), lambda i: (0, i, 0)),
            pl.BlockSpec((D, D), lambda i: (0, 0)),
            pl.BlockSpec((D, D), lambda i: (0, 0)),
            pl.BlockSpec((1, D), lambda i: (0, 0)),
        ],
        out_specs=pl.BlockSpec((BLK, D), lambda i: (i, 0)),
        out_shape=jax.ShapeDtypeStruct((N_NODES, D), jnp.float32),
    )(x, psum, pdeg, wst, wnt, b2d)


def kernel(x, edge_index, W_self, W_neigh, b):
    x = x.astype(jnp.float32)
    src = edge_index[0].astype(jnp.int32)
    dst = edge_index[1].astype(jnp.int32)

    tbl = x.reshape(N_NODES * NQ, FW)

    npad_e = EPAD - N_EDGES
    src_p = jnp.concatenate(
        [src, jnp.zeros((npad_e,), jnp.int32)]).reshape(2, 16, CHUNKS, CW)
    pad_dst = N_NODES + (
        jnp.arange(npad_e, dtype=jnp.int32) % (NPAD - N_NODES))
    dst_p = jnp.concatenate([dst, pad_dst]).reshape(2, 16, CHUNKS, CW)
    srcq_r = (
        src_p[:, :, None, :, :] * NQ
        + jnp.arange(NQ, dtype=jnp.int32)[None, None, :, None, None]
    ).reshape(2, 16, NQ * CHUNKS, CW)

    psum, pdeg = _sc_aggregate(tbl, srcq_r, dst_p)

    return _tc_combine(
        x, psum, pdeg, W_self.T, W_neigh.T, b.reshape(1, D),
    )
